# baseline (speedup 1.0000x reference)
import functools

import jax
import jax.numpy as jnp
from jax import lax
from jax.experimental import pallas as pl
from jax.experimental.pallas import tpu as pltpu
from jax.experimental.pallas import tpu_sc as plsc

N = 10000
E = 320000
D = 128
DE = 16

NC = 2
NS = 16
NW = NC * NS
GW = 16
EPW = E // NW
CPW = EPW // GW
WRS = 640
TAIL = N - WRS * (NS - 1)

EB = 4000


def _gelu(x):
    a1, a2, a3, a4, a5 = 0.254829592, -0.284496736, 1.421413741, -1.453152027, 1.061405429
    p = 0.3275911
    u = x * 0.7071067811865476
    au = jnp.abs(u)
    t = 1.0 / (1.0 + p * au)
    poly = ((((a5 * t + a4) * t + a3) * t + a2) * t + a1) * t
    erf = jnp.sign(u) * (1.0 - poly * jnp.exp(-au * au))
    return 0.5 * x * (1.0 + erf)


def _pre_body(h_ref, w_ref, b_ref, o_ref):
    o_ref[...] = jnp.dot(h_ref[...], w_ref[...],
                         preferred_element_type=jnp.float32) + b_ref[...]


def _q_body(ea_ref, w_ref, o_ref):
    o_ref[...] = jnp.dot(ea_ref[...], w_ref[...],
                         preferred_element_type=jnp.float32)


def _out_body(h_ref, a_ref, c_ref, w2_ref, b2_ref, gm_ref, bt_ref, o_ref):
    agg = a_ref[0] + a_ref[1]
    cnt = jnp.sum(c_ref[...], axis=0)
    m = jnp.dot(agg, w2_ref[...], preferred_element_type=jnp.float32)
    x = h_ref[...] + m + cnt[:, None] * b2_ref[...]
    mu = jnp.mean(x, axis=1, keepdims=True)
    xc = x - mu
    var = jnp.mean(xc * xc, axis=1, keepdims=True)
    o_ref[...] = xc * lax.rsqrt(var + 1e-5) * gm_ref[...] + bt_ref[...]


def _sc_counts(dst2):
    mesh = plsc.VectorSubcoreMesh(core_axis_name="c", subcore_axis_name="s")

    @functools.partial(
        pl.kernel,
        out_type=jax.ShapeDtypeStruct((NW * N,), jnp.float32),
        mesh=mesh,
        compiler_params=pltpu.CompilerParams(needs_layout_passes=False),
        scratch_types=[
            pltpu.VMEM((EPW,), jnp.int32),
            pltpu.VMEM((N,), jnp.float32),
        ],
    )
    def k(d_hbm, cnt_hbm, idx_v, cnt_v):
        c = lax.axis_index("c")
        s = lax.axis_index("s")
        wid = c * NS + s
        zero16 = jnp.zeros((16,), jnp.float32)
        one16 = jnp.full((16,), 1.0, jnp.float32)

        @pl.loop(0, N, step=16)
        def _(i):
            cnt_v[pl.ds(i, 16)] = zero16

        pltpu.sync_copy(d_hbm.at[wid], idx_v)

        @pl.loop(0, EPW, step=16)
        def _(i):
            plsc.addupdate_scatter(cnt_v, [idx_v[pl.ds(i, 16)]], one16)

        pltpu.sync_copy(cnt_v, cnt_hbm.at[pl.ds(wid * N, N)])

    return k(dst2)


def _sc_fused(P, Q, src3d, dst3d):
    mesh = plsc.VectorSubcoreMesh(core_axis_name="c", subcore_axis_name="s")

    @functools.partial(
        pl.kernel,
        out_type=jax.ShapeDtypeStruct((NC, N, D), jnp.float32),
        mesh=mesh,
        compiler_params=pltpu.CompilerParams(needs_layout_passes=False),
        scratch_types=[
            pltpu.VMEM((EPW,), jnp.int32),
            pltpu.VMEM((EPW,), jnp.int32),
            pltpu.VMEM((GW, D), jnp.float32),
            pltpu.VMEM((GW, D), jnp.float32),
            pltpu.VMEM((GW, D), jnp.float32),
            pltpu.VMEM((GW, D), jnp.float32),
            pltpu.VMEM_SHARED((N, D), jnp.float32),
            pltpu.SemaphoreType.DMA,
            pltpu.SemaphoreType.DMA,
            pltpu.SemaphoreType.DMA,
            pltpu.SemaphoreType.DMA,
            pltpu.SemaphoreType.DMA,
            pltpu.SemaphoreType.DMA,
        ],
    )
    def k(p_hbm, q_hbm, s_hbm, d_hbm, agg_hbm,
          sidx_v, didx_v, pb0, pb1, qb0, qb1, agg_sh,
          gs0, gs1, qs0, qs1, ss0, ss1):
        c = lax.axis_index("c")
        s = lax.axis_index("s")
        wid = c * NS + s
        zero16 = jnp.zeros((16,), jnp.float32)
        nzc = jnp.where(s == NS - 1, TAIL // GW, WRS // GW)

        @pl.loop(0, GW)
        def _(r):
            @pl.loop(0, D, step=16)
            def _(cc):
                qb0[r, pl.ds(cc, 16)] = zero16

        @pl.loop(0, nzc)
        def _(r):
            pltpu.sync_copy(qb0, agg_sh.at[pl.ds(s * WRS + r * GW, GW)])

        plsc.subcore_barrier()

        pltpu.sync_copy(s_hbm.at[wid], sidx_v)
        pltpu.sync_copy(d_hbm.at[wid], didx_v)

        def _gelu_chunk(pb, qb):
            @pl.loop(0, GW)
            def _(r):
                for t in range(D // 16):
                    sl = pl.ds(t * 16, 16)
                    qb[r, sl] = _gelu(pb[r, sl] + qb[r, sl])

        base = wid * EPW

        def _src16(j):
            return sidx_v[pl.ds(j * GW, GW)]

        def _dst16(j):
            return didx_v[pl.ds(j * GW, GW)]

        pltpu.async_copy(p_hbm.at[_src16(0)], pb0, gs0)
        pltpu.async_copy(q_hbm.at[pl.ds(base, GW)], qb0, qs0)
        pltpu.async_copy(p_hbm.at[_src16(1)], pb1, gs1)
        pltpu.async_copy(q_hbm.at[pl.ds(base + GW, GW)], qb1, qs1)

        @pl.loop(0, CPW - 1, step=2)
        def _(j):
            pltpu.make_async_copy(p_hbm.at[_src16(j)], pb0, gs0).wait()
            pltpu.make_async_copy(q_hbm.at[pl.ds(base + j * GW, GW)], qb0,
                                  qs0).wait()
            _gelu_chunk(pb0, qb0)
            pltpu.async_copy(p_hbm.at[_src16(j + 2)], pb0, gs0)
            s0 = pltpu.async_copy(qb0, agg_sh.at[_dst16(j)], ss0, add=True)

            pltpu.make_async_copy(p_hbm.at[_src16(j + 1)], pb1, gs1).wait()
            pltpu.make_async_copy(q_hbm.at[pl.ds(base + (j + 1) * GW, GW)],
                                  qb1, qs1).wait()
            _gelu_chunk(pb1, qb1)

            @pl.when(j + 3 < CPW)
            def _():
                pltpu.async_copy(p_hbm.at[_src16(j + 3)], pb1, gs1)

            s1 = pltpu.async_copy(qb1, agg_sh.at[_dst16(j + 1)], ss1,
                                  add=True)

            s0.wait()
            pltpu.async_copy(q_hbm.at[pl.ds(base + (j + 2) * GW, GW)],
                             qb0, qs0)
            s1.wait()

            @pl.when(j + 3 < CPW)
            def _():
                pltpu.async_copy(q_hbm.at[pl.ds(base + (j + 3) * GW, GW)],
                                 qb1, qs1)

        pltpu.make_async_copy(p_hbm.at[_src16(CPW - 1)], pb0, gs0).wait()
        pltpu.make_async_copy(q_hbm.at[pl.ds(base + (CPW - 1) * GW, GW)],
                              qb0, qs0).wait()
        _gelu_chunk(pb0, qb0)
        pltpu.sync_copy(qb0, agg_sh.at[_dst16(CPW - 1)], add=True)

        plsc.subcore_barrier()

        @pl.loop(0, nzc)
        def _(r):
            pltpu.sync_copy(agg_sh.at[pl.ds(s * WRS + r * GW, GW)], pb0)
            pltpu.sync_copy(pb0, agg_hbm.at[c, pl.ds(s * WRS + r * GW, GW)])

    return k(P, Q, src3d, dst3d)


def kernel(H, edge_index, edge_attr, W1, b1, W2, b2, gamma, beta):
    src = edge_index[0].astype(jnp.int32)
    dst = edge_index[1].astype(jnp.int32)
    W1a = W1[:D]
    W1b = W1[D:]
    b1r = b1.reshape(1, D)
    b2r = b2.reshape(1, D)
    gmr = gamma.reshape(1, D)
    btr = beta.reshape(1, D)

    P = pl.pallas_call(
        _pre_body,
        out_shape=jax.ShapeDtypeStruct((N, D), jnp.float32),
    )(H, W1a, b1r)

    Q = pl.pallas_call(
        _q_body,
        grid=(E // EB,),
        in_specs=[
            pl.BlockSpec((EB, DE), lambda i: (i, 0)),
            pl.BlockSpec((DE, D), lambda i: (0, 0)),
        ],
        out_specs=pl.BlockSpec((EB, D), lambda i: (i, 0)),
        out_shape=jax.ShapeDtypeStruct((E, D), jnp.float32),
    )(edge_attr, W1b)

    counts = _sc_counts(dst.reshape(NW, EPW)).reshape(NW, N)
    aggP = _sc_fused(P, Q, src.reshape(NW, EPW), dst.reshape(NW, EPW))

    out = pl.pallas_call(
        _out_body,
        out_shape=jax.ShapeDtypeStruct((N, D), jnp.float32),
    )(H, aggP, counts, W2, b2r, gmr, btr)

    return out
